# baseline (speedup 1.0000x reference)
import jax
import jax.numpy as jnp
from jax import lax
from jax.experimental import pallas as pl
from jax.experimental.pallas import tpu as pltpu
from jax.experimental.pallas import tpu_sc as plsc

VOCAB = 1000000
D = 64
B = 16384
CTX = 4
NEG = 5

NC = 2
NS = 16
NW = NC * NS
B_PER_W = B // NW
CHUNK = 64
NCHUNK = B_PER_W // CHUNK
GROUPS = CHUNK // 16

NCI = CHUNK * CTX
NNI = CHUNK * NEG

VCH = 512
NBLK = 992
OHALF = NBLK * VCH
LASTB = (VOCAB - VCH) // VCH


def _tc_tr(src_l, src_r, out):
  lt = jnp.swapaxes(src_l[...], 0, 1)
  rt = jnp.swapaxes(src_r[...], 0, 1)
  out[...] = jnp.concatenate([lt, rt], axis=1)


def _pack_table(src_t):
  return pl.pallas_call(
      _tc_tr,
      grid=(NBLK,),
      in_specs=[
          pl.BlockSpec((D, VCH), lambda i: (0, i)),
          pl.BlockSpec((D, VCH), lambda i: (0, jnp.minimum(NBLK + i, LASTB))),
      ],
      out_specs=pl.BlockSpec((VCH, 128), lambda i: (i, 0)),
      out_shape=jax.ShapeDtypeStruct((OHALF, 128), jnp.float32),
  )(src_t, src_t)


def _body(ctx_idx_hbm, cen_idx_hbm, neg_idx_hbm, ctx_emb_hbm, cen_emb_hbm,
          pos_hbm, neg_hbm,
          idx_ctx, idx_cen, idx_neg, pr_ctx, pr_cen, pr_neg,
          rows_ctx, rows_cen, rows_neg, pos_v, neg_v, sem):
  wid = lax.axis_index("s") * NC + lax.axis_index("c")
  base = wid * B_PER_W

  lanes = lax.iota(jnp.int32, 16)
  oh = jnp.int32(OHALF)
  c64 = jnp.int32(64)
  zero = jnp.int32(0)

  for c in range(NCHUNK):
    b0 = base + c * CHUNK
    pltpu.sync_copy(ctx_idx_hbm.at[pl.ds(b0 * CTX, NCI)], idx_ctx)
    pltpu.sync_copy(cen_idx_hbm.at[pl.ds(b0, CHUNK)], idx_cen)
    pltpu.sync_copy(neg_idx_hbm.at[pl.ds(b0 * NEG, NNI)], idx_neg)

    def pair_into(dst, src, n):
      def sbody(k, _):
        iv = src[pl.ds(k * 16, 16)]
        dst[pl.ds(k * 16, 16)] = jnp.where(iv >= oh, iv - oh, iv)
        return 0
      lax.fori_loop(0, n // 16, sbody, 0)
    pair_into(pr_ctx, idx_ctx, NCI)
    pair_into(pr_cen, idx_cen, CHUNK)
    pair_into(pr_neg, idx_neg, NNI)

    cps = []
    for k in range(NCI // 128):
      cps.append(pltpu.make_async_copy(
          ctx_emb_hbm.at[pr_ctx.at[pl.ds(k * 128, 128)]],
          rows_ctx.at[pl.ds(k * 128, 128)], sem))
    cps.append(pltpu.make_async_copy(
        cen_emb_hbm.at[pr_cen], rows_cen, sem))
    for k in range(NNI // 64):
      cps.append(pltpu.make_async_copy(
          cen_emb_hbm.at[pr_neg.at[pl.ds(k * 64, 64)]],
          rows_neg.at[pl.ds(k * 64, 64)], sem))
    for cp in cps:
      cp.start()
    for cp in cps:
      cp.wait()

    def group_body(g, _):
      bl = g * 16 + lanes
      ctx_rows = bl * CTX
      neg_rows = bl * NEG

      def half(iref, pos_vec):
        v = plsc.load_gather(iref, [pos_vec])
        return jnp.where(v >= oh, c64, zero)

      cb_c0 = half(idx_ctx, ctx_rows)
      cb_c1 = half(idx_ctx, ctx_rows + 1)
      cb_c2 = half(idx_ctx, ctx_rows + 2)
      cb_c3 = half(idx_ctx, ctx_rows + 3)
      cb_u = half(idx_cen, bl)
      cb_n0 = half(idx_neg, neg_rows)
      cb_n1 = half(idx_neg, neg_rows + 1)
      cb_n2 = half(idx_neg, neg_rows + 2)
      cb_n3 = half(idx_neg, neg_rows + 3)
      cb_n4 = half(idx_neg, neg_rows + 4)

      def d_body(d, acc):
        pos_a, n0, n1, n2, n3, n4 = acc
        v = plsc.load_gather(rows_ctx, [ctx_rows, cb_c0 + d])
        v = v + plsc.load_gather(rows_ctx, [ctx_rows + 1, cb_c1 + d])
        v = v + plsc.load_gather(rows_ctx, [ctx_rows + 2, cb_c2 + d])
        v = v + plsc.load_gather(rows_ctx, [ctx_rows + 3, cb_c3 + d])
        u = plsc.load_gather(rows_cen, [bl, cb_u + d])
        pos_a = pos_a + v * u
        n0 = n0 + v * plsc.load_gather(rows_neg, [neg_rows, cb_n0 + d])
        n1 = n1 + v * plsc.load_gather(rows_neg, [neg_rows + 1, cb_n1 + d])
        n2 = n2 + v * plsc.load_gather(rows_neg, [neg_rows + 2, cb_n2 + d])
        n3 = n3 + v * plsc.load_gather(rows_neg, [neg_rows + 3, cb_n3 + d])
        n4 = n4 + v * plsc.load_gather(rows_neg, [neg_rows + 4, cb_n4 + d])
        return pos_a, n0, n1, n2, n3, n4

      z = jnp.zeros((16,), jnp.float32)
      pos_a, n0, n1, n2, n3, n4 = lax.fori_loop(
          0, D, d_body, (z, z, z, z, z, z))

      quarter = jnp.float32(0.25)
      pos_v[pl.ds(g * 16, 16)] = pos_a * quarter
      plsc.store_scatter(neg_v, [neg_rows], n0 * quarter)
      plsc.store_scatter(neg_v, [neg_rows + 1], n1 * quarter)
      plsc.store_scatter(neg_v, [neg_rows + 2], n2 * quarter)
      plsc.store_scatter(neg_v, [neg_rows + 3], n3 * quarter)
      plsc.store_scatter(neg_v, [neg_rows + 4], n4 * quarter)
      return 0

    lax.fori_loop(0, GROUPS, group_body, 0)

    pltpu.sync_copy(pos_v, pos_hbm.at[pl.ds(b0, CHUNK)])
    pltpu.sync_copy(neg_v, neg_hbm.at[pl.ds(b0 * NEG, NNI)])


@jax.jit
def _cbow_sc(ctx_idx, cen_idx, neg_idx, ctx_t, cen_t):
  ctx_emb2 = _pack_table(ctx_t)
  cen_emb2 = _pack_table(cen_t)

  mesh = plsc.VectorSubcoreMesh(core_axis_name="c", subcore_axis_name="s")
  kfn = pl.kernel(
      _body,
      out_type=(
          jax.ShapeDtypeStruct((B,), jnp.float32),
          jax.ShapeDtypeStruct((B * NEG,), jnp.float32),
      ),
      mesh=mesh,
      compiler_params=pltpu.CompilerParams(
          needs_layout_passes=False, use_tc_tiling_on_sc=True),
      scratch_types=[
          pltpu.VMEM((NCI,), jnp.int32),
          pltpu.VMEM((CHUNK,), jnp.int32),
          pltpu.VMEM((NNI,), jnp.int32),
          pltpu.VMEM((NCI,), jnp.int32),
          pltpu.VMEM((CHUNK,), jnp.int32),
          pltpu.VMEM((NNI,), jnp.int32),
          pltpu.VMEM((NCI, 128), jnp.float32),
          pltpu.VMEM((CHUNK, 128), jnp.float32),
          pltpu.VMEM((NNI, 128), jnp.float32),
          pltpu.VMEM((CHUNK,), jnp.float32),
          pltpu.VMEM((NNI,), jnp.float32),
          pltpu.SemaphoreType.DMA,
      ],
  )
  return kfn(ctx_idx, cen_idx, neg_idx, ctx_emb2, cen_emb2)


def kernel(context_words, center_words, negative_samples, context_emb,
           center_emb):
  ctx_idx = context_words.reshape(-1).astype(jnp.int32)
  cen_idx = center_words.astype(jnp.int32)
  neg_idx = negative_samples.reshape(-1).astype(jnp.int32)
  pos, neg = _cbow_sc(ctx_idx, cen_idx, neg_idx, context_emb.T, center_emb.T)
  return pos, neg.reshape(B, NEG)


# speedup vs baseline: 1.7907x; 1.7907x over previous
import jax
import jax.numpy as jnp
from jax import lax
from jax.experimental import pallas as pl
from jax.experimental.pallas import tpu as pltpu
from jax.experimental.pallas import tpu_sc as plsc

VOCAB = 1000000
D = 64
B = 16384
CTX = 4
NEG = 5

NC = 2
NS = 16
NW = NC * NS
B_PER_W = B // NW
CHUNK = 64
NCHUNK = B_PER_W // CHUNK
GROUPS = CHUNK // 16

NCI = CHUNK * CTX
NNI = CHUNK * NEG

VCH = 2048
NBLK = 248
OHALF = NBLK * VCH
LASTB = (VOCAB + VCH - 1) // VCH - 1


def _tc_tr(src_l, src_r, out):
  lt = jnp.swapaxes(src_l[...], 0, 1)
  rt = jnp.swapaxes(src_r[...], 0, 1)
  out[...] = jnp.concatenate([lt, rt], axis=1)


def _pack_table(src_t):
  return pl.pallas_call(
      _tc_tr,
      grid=(NBLK,),
      in_specs=[
          pl.BlockSpec((D, VCH), lambda i: (0, i)),
          pl.BlockSpec((D, VCH), lambda i: (0, jnp.minimum(NBLK + i, LASTB))),
      ],
      out_specs=pl.BlockSpec((VCH, 128), lambda i: (i, 0)),
      out_shape=jax.ShapeDtypeStruct((OHALF, 128), jnp.float32),
  )(src_t, src_t)


def _body(ctx_idx_hbm, cen_idx_hbm, neg_idx_hbm, ctx_emb_hbm, cen_emb_hbm,
          pos_hbm, neg_hbm,
          idx_ctx, idx_cen, idx_neg, pr_ctx, pr_cen, pr_neg,
          rows_ctx, rows_cen, rows_neg, pos_v, neg_v, sem):
  wid = lax.axis_index("s") * NC + lax.axis_index("c")
  base = wid * B_PER_W

  lanes = lax.iota(jnp.int32, 16)
  oh = jnp.int32(OHALF)
  c64 = jnp.int32(64)
  zero = jnp.int32(0)

  for c in range(NCHUNK):
    b0 = base + c * CHUNK
    pltpu.sync_copy(ctx_idx_hbm.at[pl.ds(b0 * CTX, NCI)], idx_ctx)
    pltpu.sync_copy(cen_idx_hbm.at[pl.ds(b0, CHUNK)], idx_cen)
    pltpu.sync_copy(neg_idx_hbm.at[pl.ds(b0 * NEG, NNI)], idx_neg)

    def pair_into(dst, src, n):
      def sbody(k, _):
        iv = src[pl.ds(k * 16, 16)]
        dst[pl.ds(k * 16, 16)] = jnp.where(iv >= oh, iv - oh, iv)
        return 0
      lax.fori_loop(0, n // 16, sbody, 0)
    pair_into(pr_ctx, idx_ctx, NCI)
    pair_into(pr_cen, idx_cen, CHUNK)
    pair_into(pr_neg, idx_neg, NNI)

    cps = []
    for k in range(NCI // 128):
      cps.append(pltpu.make_async_copy(
          ctx_emb_hbm.at[pr_ctx.at[pl.ds(k * 128, 128)]],
          rows_ctx.at[pl.ds(k * 128, 128)], sem))
    cps.append(pltpu.make_async_copy(
        cen_emb_hbm.at[pr_cen], rows_cen, sem))
    for k in range(NNI // 64):
      cps.append(pltpu.make_async_copy(
          cen_emb_hbm.at[pr_neg.at[pl.ds(k * 64, 64)]],
          rows_neg.at[pl.ds(k * 64, 64)], sem))
    for cp in cps:
      cp.start()
    for cp in cps:
      cp.wait()

    def group_body(g, _):
      bl = g * 16 + lanes
      ctx_rows = bl * CTX
      neg_rows = bl * NEG

      def half(iref, pos_vec):
        v = plsc.load_gather(iref, [pos_vec])
        return jnp.where(v >= oh, c64, zero)

      cb_c0 = half(idx_ctx, ctx_rows)
      cb_c1 = half(idx_ctx, ctx_rows + 1)
      cb_c2 = half(idx_ctx, ctx_rows + 2)
      cb_c3 = half(idx_ctx, ctx_rows + 3)
      cb_u = half(idx_cen, bl)
      cb_n0 = half(idx_neg, neg_rows)
      cb_n1 = half(idx_neg, neg_rows + 1)
      cb_n2 = half(idx_neg, neg_rows + 2)
      cb_n3 = half(idx_neg, neg_rows + 3)
      cb_n4 = half(idx_neg, neg_rows + 4)

      def d_body(d, acc):
        pos_a, n0, n1, n2, n3, n4 = acc
        v = plsc.load_gather(rows_ctx, [ctx_rows, cb_c0 + d])
        v = v + plsc.load_gather(rows_ctx, [ctx_rows + 1, cb_c1 + d])
        v = v + plsc.load_gather(rows_ctx, [ctx_rows + 2, cb_c2 + d])
        v = v + plsc.load_gather(rows_ctx, [ctx_rows + 3, cb_c3 + d])
        u = plsc.load_gather(rows_cen, [bl, cb_u + d])
        pos_a = pos_a + v * u
        n0 = n0 + v * plsc.load_gather(rows_neg, [neg_rows, cb_n0 + d])
        n1 = n1 + v * plsc.load_gather(rows_neg, [neg_rows + 1, cb_n1 + d])
        n2 = n2 + v * plsc.load_gather(rows_neg, [neg_rows + 2, cb_n2 + d])
        n3 = n3 + v * plsc.load_gather(rows_neg, [neg_rows + 3, cb_n3 + d])
        n4 = n4 + v * plsc.load_gather(rows_neg, [neg_rows + 4, cb_n4 + d])
        return pos_a, n0, n1, n2, n3, n4

      z = jnp.zeros((16,), jnp.float32)
      pos_a, n0, n1, n2, n3, n4 = lax.fori_loop(
          0, D, d_body, (z, z, z, z, z, z))

      quarter = jnp.float32(0.25)
      pos_v[pl.ds(g * 16, 16)] = pos_a * quarter
      plsc.store_scatter(neg_v, [neg_rows], n0 * quarter)
      plsc.store_scatter(neg_v, [neg_rows + 1], n1 * quarter)
      plsc.store_scatter(neg_v, [neg_rows + 2], n2 * quarter)
      plsc.store_scatter(neg_v, [neg_rows + 3], n3 * quarter)
      plsc.store_scatter(neg_v, [neg_rows + 4], n4 * quarter)
      return 0

    lax.fori_loop(0, GROUPS, group_body, 0)

    pltpu.sync_copy(pos_v, pos_hbm.at[pl.ds(b0, CHUNK)])
    pltpu.sync_copy(neg_v, neg_hbm.at[pl.ds(b0 * NEG, NNI)])


@jax.jit
def _cbow_sc(ctx_idx, cen_idx, neg_idx, ctx_t, cen_t):
  ctx_emb2 = _pack_table(ctx_t)
  cen_emb2 = _pack_table(cen_t)

  mesh = plsc.VectorSubcoreMesh(core_axis_name="c", subcore_axis_name="s")
  kfn = pl.kernel(
      _body,
      out_type=(
          jax.ShapeDtypeStruct((B,), jnp.float32),
          jax.ShapeDtypeStruct((B * NEG,), jnp.float32),
      ),
      mesh=mesh,
      compiler_params=pltpu.CompilerParams(
          needs_layout_passes=False, use_tc_tiling_on_sc=True),
      scratch_types=[
          pltpu.VMEM((NCI,), jnp.int32),
          pltpu.VMEM((CHUNK,), jnp.int32),
          pltpu.VMEM((NNI,), jnp.int32),
          pltpu.VMEM((NCI,), jnp.int32),
          pltpu.VMEM((CHUNK,), jnp.int32),
          pltpu.VMEM((NNI,), jnp.int32),
          pltpu.VMEM((NCI, 128), jnp.float32),
          pltpu.VMEM((CHUNK, 128), jnp.float32),
          pltpu.VMEM((NNI, 128), jnp.float32),
          pltpu.VMEM((CHUNK,), jnp.float32),
          pltpu.VMEM((NNI,), jnp.float32),
          pltpu.SemaphoreType.DMA,
      ],
  )
  return kfn(ctx_idx, cen_idx, neg_idx, ctx_emb2, cen_emb2)


def kernel(context_words, center_words, negative_samples, context_emb,
           center_emb):
  ctx_idx = context_words.reshape(-1).astype(jnp.int32)
  cen_idx = center_words.astype(jnp.int32)
  neg_idx = negative_samples.reshape(-1).astype(jnp.int32)
  pos, neg = _cbow_sc(ctx_idx, cen_idx, neg_idx, context_emb.T, center_emb.T)
  return pos, neg.reshape(B, NEG)


# speedup vs baseline: 2.2940x; 1.2811x over previous
import jax
import jax.numpy as jnp
from jax import lax
from jax.experimental import pallas as pl
from jax.experimental.pallas import tpu as pltpu
from jax.experimental.pallas import tpu_sc as plsc

VOCAB = 1000000
D = 64
B = 16384
CTX = 4
NEG = 5

NC = 2
NS = 16
NW = NC * NS
B_PER_W = B // NW
CHUNK = 64
NCHUNK = B_PER_W // CHUNK
GROUPS = CHUNK // 16

NCI = CHUNK * CTX
NNI = CHUNK * NEG

VCH = 8192
NBLK = 62
OHALF = NBLK * VCH
LASTB = (VOCAB + VCH - 1) // VCH - 1


def _tc_tr(src_l, src_r, out):
  lt = jnp.swapaxes(src_l[...], 0, 1)
  rt = jnp.swapaxes(src_r[...], 0, 1)
  out[...] = jnp.concatenate([lt, rt], axis=1)


def _pack_table(src_t):
  return pl.pallas_call(
      _tc_tr,
      grid=(NBLK,),
      in_specs=[
          pl.BlockSpec((D, VCH), lambda i: (0, i)),
          pl.BlockSpec((D, VCH), lambda i: (0, jnp.minimum(NBLK + i, LASTB))),
      ],
      out_specs=pl.BlockSpec((VCH, 128), lambda i: (i, 0)),
      out_shape=jax.ShapeDtypeStruct((OHALF, 128), jnp.float32),
  )(src_t, src_t)


def _body(ctx_idx_hbm, cen_idx_hbm, neg_idx_hbm, ctx_emb_hbm, cen_emb_hbm,
          pos_hbm, neg_hbm,
          idx_ctx, idx_cen, idx_neg, pr_ctx, pr_cen, pr_neg,
          rows_ctx, rows_cen, rows_neg, pos_v, neg_v, sem):
  wid = lax.axis_index("s") * NC + lax.axis_index("c")
  base = wid * B_PER_W

  lanes = lax.iota(jnp.int32, 16)
  oh = jnp.int32(OHALF)
  c64 = jnp.int32(64)
  zero = jnp.int32(0)

  for c in range(NCHUNK):
    b0 = base + c * CHUNK
    pltpu.sync_copy(ctx_idx_hbm.at[pl.ds(b0 * CTX, NCI)], idx_ctx)
    pltpu.sync_copy(cen_idx_hbm.at[pl.ds(b0, CHUNK)], idx_cen)
    pltpu.sync_copy(neg_idx_hbm.at[pl.ds(b0 * NEG, NNI)], idx_neg)

    def pair_into(dst, src, n):
      def sbody(k, _):
        iv = src[pl.ds(k * 16, 16)]
        dst[pl.ds(k * 16, 16)] = jnp.where(iv >= oh, iv - oh, iv)
        return 0
      lax.fori_loop(0, n // 16, sbody, 0)
    pair_into(pr_ctx, idx_ctx, NCI)
    pair_into(pr_cen, idx_cen, CHUNK)
    pair_into(pr_neg, idx_neg, NNI)

    cps = []
    for k in range(NCI // 128):
      cps.append(pltpu.make_async_copy(
          ctx_emb_hbm.at[pr_ctx.at[pl.ds(k * 128, 128)]],
          rows_ctx.at[pl.ds(k * 128, 128)], sem))
    cps.append(pltpu.make_async_copy(
        cen_emb_hbm.at[pr_cen], rows_cen, sem))
    for k in range(NNI // 64):
      cps.append(pltpu.make_async_copy(
          cen_emb_hbm.at[pr_neg.at[pl.ds(k * 64, 64)]],
          rows_neg.at[pl.ds(k * 64, 64)], sem))
    for cp in cps:
      cp.start()
    for cp in cps:
      cp.wait()

    def group_body(g, _):
      bl = g * 16 + lanes
      ctx_rows = bl * CTX
      neg_rows = bl * NEG

      def half(iref, pos_vec):
        v = plsc.load_gather(iref, [pos_vec])
        return jnp.where(v >= oh, c64, zero)

      cb_c0 = half(idx_ctx, ctx_rows)
      cb_c1 = half(idx_ctx, ctx_rows + 1)
      cb_c2 = half(idx_ctx, ctx_rows + 2)
      cb_c3 = half(idx_ctx, ctx_rows + 3)
      cb_u = half(idx_cen, bl)
      cb_n0 = half(idx_neg, neg_rows)
      cb_n1 = half(idx_neg, neg_rows + 1)
      cb_n2 = half(idx_neg, neg_rows + 2)
      cb_n3 = half(idx_neg, neg_rows + 3)
      cb_n4 = half(idx_neg, neg_rows + 4)

      def d_body(d, acc):
        pos_a, n0, n1, n2, n3, n4 = acc
        v = plsc.load_gather(rows_ctx, [ctx_rows, cb_c0 + d])
        v = v + plsc.load_gather(rows_ctx, [ctx_rows + 1, cb_c1 + d])
        v = v + plsc.load_gather(rows_ctx, [ctx_rows + 2, cb_c2 + d])
        v = v + plsc.load_gather(rows_ctx, [ctx_rows + 3, cb_c3 + d])
        u = plsc.load_gather(rows_cen, [bl, cb_u + d])
        pos_a = pos_a + v * u
        n0 = n0 + v * plsc.load_gather(rows_neg, [neg_rows, cb_n0 + d])
        n1 = n1 + v * plsc.load_gather(rows_neg, [neg_rows + 1, cb_n1 + d])
        n2 = n2 + v * plsc.load_gather(rows_neg, [neg_rows + 2, cb_n2 + d])
        n3 = n3 + v * plsc.load_gather(rows_neg, [neg_rows + 3, cb_n3 + d])
        n4 = n4 + v * plsc.load_gather(rows_neg, [neg_rows + 4, cb_n4 + d])
        return pos_a, n0, n1, n2, n3, n4

      z = jnp.zeros((16,), jnp.float32)
      pos_a, n0, n1, n2, n3, n4 = lax.fori_loop(
          0, D, d_body, (z, z, z, z, z, z))

      quarter = jnp.float32(0.25)
      pos_v[pl.ds(g * 16, 16)] = pos_a * quarter
      plsc.store_scatter(neg_v, [neg_rows], n0 * quarter)
      plsc.store_scatter(neg_v, [neg_rows + 1], n1 * quarter)
      plsc.store_scatter(neg_v, [neg_rows + 2], n2 * quarter)
      plsc.store_scatter(neg_v, [neg_rows + 3], n3 * quarter)
      plsc.store_scatter(neg_v, [neg_rows + 4], n4 * quarter)
      return 0

    lax.fori_loop(0, GROUPS, group_body, 0)

    pltpu.sync_copy(pos_v, pos_hbm.at[pl.ds(b0, CHUNK)])
    pltpu.sync_copy(neg_v, neg_hbm.at[pl.ds(b0 * NEG, NNI)])


@jax.jit
def _cbow_sc(ctx_idx, cen_idx, neg_idx, ctx_t, cen_t):
  ctx_emb2 = _pack_table(ctx_t)
  cen_emb2 = _pack_table(cen_t)

  mesh = plsc.VectorSubcoreMesh(core_axis_name="c", subcore_axis_name="s")
  kfn = pl.kernel(
      _body,
      out_type=(
          jax.ShapeDtypeStruct((B,), jnp.float32),
          jax.ShapeDtypeStruct((B * NEG,), jnp.float32),
      ),
      mesh=mesh,
      compiler_params=pltpu.CompilerParams(
          needs_layout_passes=False, use_tc_tiling_on_sc=True),
      scratch_types=[
          pltpu.VMEM((NCI,), jnp.int32),
          pltpu.VMEM((CHUNK,), jnp.int32),
          pltpu.VMEM((NNI,), jnp.int32),
          pltpu.VMEM((NCI,), jnp.int32),
          pltpu.VMEM((CHUNK,), jnp.int32),
          pltpu.VMEM((NNI,), jnp.int32),
          pltpu.VMEM((NCI, 128), jnp.float32),
          pltpu.VMEM((CHUNK, 128), jnp.float32),
          pltpu.VMEM((NNI, 128), jnp.float32),
          pltpu.VMEM((CHUNK,), jnp.float32),
          pltpu.VMEM((NNI,), jnp.float32),
          pltpu.SemaphoreType.DMA,
      ],
  )
  return kfn(ctx_idx, cen_idx, neg_idx, ctx_emb2, cen_emb2)


def kernel(context_words, center_words, negative_samples, context_emb,
           center_emb):
  ctx_idx = context_words.reshape(-1).astype(jnp.int32)
  cen_idx = center_words.astype(jnp.int32)
  neg_idx = negative_samples.reshape(-1).astype(jnp.int32)
  pos, neg = _cbow_sc(ctx_idx, cen_idx, neg_idx, context_emb.T, center_emb.T)
  return pos, neg.reshape(B, NEG)


# speedup vs baseline: 2.3837x; 1.0391x over previous
import jax
import jax.numpy as jnp
from jax import lax
from jax.experimental import pallas as pl
from jax.experimental.pallas import tpu as pltpu
from jax.experimental.pallas import tpu_sc as plsc

VOCAB = 1000000
D = 64
B = 16384
CTX = 4
NEG = 5

NC = 2
NS = 16
NW = NC * NS
B_PER_W = B // NW
CHUNK = 64
NCHUNK = B_PER_W // CHUNK
GROUPS = CHUNK // 16

NCI = CHUNK * CTX
NNI = CHUNK * NEG

VCH = 16384
NBLK = 31
OHALF = NBLK * VCH
LASTB = (VOCAB + VCH - 1) // VCH - 1


def _tc_tr(src_l, src_r, out):
  lt = jnp.swapaxes(src_l[...], 0, 1)
  rt = jnp.swapaxes(src_r[...], 0, 1)
  out[...] = jnp.concatenate([lt, rt], axis=1)


def _pack_table(src_t):
  return pl.pallas_call(
      _tc_tr,
      grid=(NBLK,),
      in_specs=[
          pl.BlockSpec((D, VCH), lambda i: (0, i)),
          pl.BlockSpec((D, VCH), lambda i: (0, jnp.minimum(NBLK + i, LASTB))),
      ],
      out_specs=pl.BlockSpec((VCH, 128), lambda i: (i, 0)),
      out_shape=jax.ShapeDtypeStruct((OHALF, 128), jnp.float32),
  )(src_t, src_t)


def _body(ctx_idx_hbm, cen_idx_hbm, neg_idx_hbm, ctx_emb_hbm, cen_emb_hbm,
          pos_hbm, neg_hbm,
          idx_ctx, idx_cen, idx_neg, pr_ctx, pr_cen, pr_neg,
          rows_ctx, rows_cen, rows_neg, pos_v, neg_v, sem):
  wid = lax.axis_index("s") * NC + lax.axis_index("c")
  base = wid * B_PER_W

  lanes = lax.iota(jnp.int32, 16)
  oh = jnp.int32(OHALF)
  c64 = jnp.int32(64)
  zero = jnp.int32(0)

  for c in range(NCHUNK):
    b0 = base + c * CHUNK
    pltpu.sync_copy(ctx_idx_hbm.at[pl.ds(b0 * CTX, NCI)], idx_ctx)
    pltpu.sync_copy(cen_idx_hbm.at[pl.ds(b0, CHUNK)], idx_cen)
    pltpu.sync_copy(neg_idx_hbm.at[pl.ds(b0 * NEG, NNI)], idx_neg)

    def pair_into(dst, src, n):
      def sbody(k, _):
        iv = src[pl.ds(k * 16, 16)]
        dst[pl.ds(k * 16, 16)] = jnp.where(iv >= oh, iv - oh, iv)
        return 0
      lax.fori_loop(0, n // 16, sbody, 0)
    pair_into(pr_ctx, idx_ctx, NCI)
    pair_into(pr_cen, idx_cen, CHUNK)
    pair_into(pr_neg, idx_neg, NNI)

    cps = []
    for k in range(NCI // 128):
      cps.append(pltpu.make_async_copy(
          ctx_emb_hbm.at[pr_ctx.at[pl.ds(k * 128, 128)]],
          rows_ctx.at[pl.ds(k * 128, 128)], sem))
    cps.append(pltpu.make_async_copy(
        cen_emb_hbm.at[pr_cen], rows_cen, sem))
    for k in range(NNI // 64):
      cps.append(pltpu.make_async_copy(
          cen_emb_hbm.at[pr_neg.at[pl.ds(k * 64, 64)]],
          rows_neg.at[pl.ds(k * 64, 64)], sem))
    for cp in cps:
      cp.start()
    for cp in cps:
      cp.wait()

    def group_body(g, _):
      bl = g * 16 + lanes
      ctx_rows = bl * CTX
      neg_rows = bl * NEG

      def half(iref, pos_vec):
        v = plsc.load_gather(iref, [pos_vec])
        return jnp.where(v >= oh, c64, zero)

      cb_c0 = half(idx_ctx, ctx_rows)
      cb_c1 = half(idx_ctx, ctx_rows + 1)
      cb_c2 = half(idx_ctx, ctx_rows + 2)
      cb_c3 = half(idx_ctx, ctx_rows + 3)
      cb_u = half(idx_cen, bl)
      cb_n0 = half(idx_neg, neg_rows)
      cb_n1 = half(idx_neg, neg_rows + 1)
      cb_n2 = half(idx_neg, neg_rows + 2)
      cb_n3 = half(idx_neg, neg_rows + 3)
      cb_n4 = half(idx_neg, neg_rows + 4)

      def d_body(d, acc):
        pos_a, n0, n1, n2, n3, n4 = acc
        v = plsc.load_gather(rows_ctx, [ctx_rows, cb_c0 + d])
        v = v + plsc.load_gather(rows_ctx, [ctx_rows + 1, cb_c1 + d])
        v = v + plsc.load_gather(rows_ctx, [ctx_rows + 2, cb_c2 + d])
        v = v + plsc.load_gather(rows_ctx, [ctx_rows + 3, cb_c3 + d])
        u = plsc.load_gather(rows_cen, [bl, cb_u + d])
        pos_a = pos_a + v * u
        n0 = n0 + v * plsc.load_gather(rows_neg, [neg_rows, cb_n0 + d])
        n1 = n1 + v * plsc.load_gather(rows_neg, [neg_rows + 1, cb_n1 + d])
        n2 = n2 + v * plsc.load_gather(rows_neg, [neg_rows + 2, cb_n2 + d])
        n3 = n3 + v * plsc.load_gather(rows_neg, [neg_rows + 3, cb_n3 + d])
        n4 = n4 + v * plsc.load_gather(rows_neg, [neg_rows + 4, cb_n4 + d])
        return pos_a, n0, n1, n2, n3, n4

      z = jnp.zeros((16,), jnp.float32)
      pos_a, n0, n1, n2, n3, n4 = lax.fori_loop(
          0, D, d_body, (z, z, z, z, z, z))

      quarter = jnp.float32(0.25)
      pos_v[pl.ds(g * 16, 16)] = pos_a * quarter
      plsc.store_scatter(neg_v, [neg_rows], n0 * quarter)
      plsc.store_scatter(neg_v, [neg_rows + 1], n1 * quarter)
      plsc.store_scatter(neg_v, [neg_rows + 2], n2 * quarter)
      plsc.store_scatter(neg_v, [neg_rows + 3], n3 * quarter)
      plsc.store_scatter(neg_v, [neg_rows + 4], n4 * quarter)
      return 0

    lax.fori_loop(0, GROUPS, group_body, 0)

    pltpu.sync_copy(pos_v, pos_hbm.at[pl.ds(b0, CHUNK)])
    pltpu.sync_copy(neg_v, neg_hbm.at[pl.ds(b0 * NEG, NNI)])


@jax.jit
def _cbow_sc(ctx_idx, cen_idx, neg_idx, ctx_t, cen_t):
  ctx_emb2 = _pack_table(ctx_t)
  cen_emb2 = _pack_table(cen_t)

  mesh = plsc.VectorSubcoreMesh(core_axis_name="c", subcore_axis_name="s")
  kfn = pl.kernel(
      _body,
      out_type=(
          jax.ShapeDtypeStruct((B,), jnp.float32),
          jax.ShapeDtypeStruct((B * NEG,), jnp.float32),
      ),
      mesh=mesh,
      compiler_params=pltpu.CompilerParams(
          needs_layout_passes=False, use_tc_tiling_on_sc=True),
      scratch_types=[
          pltpu.VMEM((NCI,), jnp.int32),
          pltpu.VMEM((CHUNK,), jnp.int32),
          pltpu.VMEM((NNI,), jnp.int32),
          pltpu.VMEM((NCI,), jnp.int32),
          pltpu.VMEM((CHUNK,), jnp.int32),
          pltpu.VMEM((NNI,), jnp.int32),
          pltpu.VMEM((NCI, 128), jnp.float32),
          pltpu.VMEM((CHUNK, 128), jnp.float32),
          pltpu.VMEM((NNI, 128), jnp.float32),
          pltpu.VMEM((CHUNK,), jnp.float32),
          pltpu.VMEM((NNI,), jnp.float32),
          pltpu.SemaphoreType.DMA,
      ],
  )
  return kfn(ctx_idx, cen_idx, neg_idx, ctx_emb2, cen_emb2)


def kernel(context_words, center_words, negative_samples, context_emb,
           center_emb):
  ctx_idx = context_words.reshape(-1).astype(jnp.int32)
  cen_idx = center_words.astype(jnp.int32)
  neg_idx = negative_samples.reshape(-1).astype(jnp.int32)
  pos, neg = _cbow_sc(ctx_idx, cen_idx, neg_idx, context_emb.T, center_emb.T)
  return pos, neg.reshape(B, NEG)


# speedup vs baseline: 2.4716x; 1.0369x over previous
import jax
import jax.numpy as jnp
from jax import lax
from jax.experimental import pallas as pl
from jax.experimental.pallas import tpu as pltpu
from jax.experimental.pallas import tpu_sc as plsc

VOCAB = 1000000
D = 64
B = 16384
CTX = 4
NEG = 5

NC = 2
NS = 16
NW = NC * NS
B_PER_W = B // NW
CHUNK = 64
NCHUNK = B_PER_W // CHUNK
GROUPS = CHUNK // 16

NCI = CHUNK * CTX
NNI = CHUNK * NEG

VCH = 16384
NBLK = 31
OHALF = NBLK * VCH
LASTB = (VOCAB + VCH - 1) // VCH - 1


def _tc_tr(src_l, src_r, out):
  lt = jnp.swapaxes(src_l[...], 0, 1)
  rt = jnp.swapaxes(src_r[...], 0, 1)
  out[...] = jnp.concatenate([lt, rt], axis=1)


def _pack_table(src_t):
  return pl.pallas_call(
      _tc_tr,
      grid=(NBLK,),
      in_specs=[
          pl.BlockSpec((D, VCH), lambda i: (0, i)),
          pl.BlockSpec((D, VCH), lambda i: (0, jnp.minimum(NBLK + i, LASTB))),
      ],
      out_specs=pl.BlockSpec((VCH, 128), lambda i: (i, 0)),
      out_shape=jax.ShapeDtypeStruct((OHALF, 128), jnp.float32),
  )(src_t, src_t)


def _body(ctx_idx_hbm, cen_idx_hbm, neg_idx_hbm, ctx_emb_hbm, cen_emb_hbm,
          pos_hbm, neg_hbm,
          idx_ctx, idx_cen, idx_neg, pr_ctx, pr_cen, pr_neg,
          rows_ctx, rows_cen, rows_neg, pos_v, neg_v, sem):
  wid = lax.axis_index("s") * NC + lax.axis_index("c")
  base = wid * B_PER_W

  lanes = lax.iota(jnp.int32, 16)
  oh = jnp.int32(OHALF)
  c64 = jnp.int32(64)
  zero = jnp.int32(0)

  for c in range(NCHUNK):
    b0 = base + c * CHUNK
    pltpu.sync_copy(ctx_idx_hbm.at[pl.ds(b0 * CTX, NCI)], idx_ctx)
    pltpu.sync_copy(cen_idx_hbm.at[pl.ds(b0, CHUNK)], idx_cen)
    pltpu.sync_copy(neg_idx_hbm.at[pl.ds(b0 * NEG, NNI)], idx_neg)

    def pair_into(dst, src, n):
      def sbody(k, _):
        iv = src[pl.ds(k * 16, 16)]
        dst[pl.ds(k * 16, 16)] = jnp.where(iv >= oh, iv - oh, iv)
        return 0
      lax.fori_loop(0, n // 16, sbody, 0)
    pair_into(pr_ctx, idx_ctx, NCI)
    pair_into(pr_cen, idx_cen, CHUNK)
    pair_into(pr_neg, idx_neg, NNI)

    cps = []
    for k in range(NCI // 128):
      cps.append(pltpu.make_async_copy(
          ctx_emb_hbm.at[pr_ctx.at[pl.ds(k * 128, 128)]],
          rows_ctx.at[pl.ds(k * 128, 128)], sem))
    cps.append(pltpu.make_async_copy(
        cen_emb_hbm.at[pr_cen], rows_cen, sem))
    for k in range(NNI // 64):
      cps.append(pltpu.make_async_copy(
          cen_emb_hbm.at[pr_neg.at[pl.ds(k * 64, 64)]],
          rows_neg.at[pl.ds(k * 64, 64)], sem))
    for cp in cps:
      cp.start()
    for cp in cps:
      cp.wait()

    def group_body(g, _):
      bl = g * 16 + lanes
      ctx_rows = bl * CTX
      neg_rows = bl * NEG

      def half(iref, pos_vec):
        v = plsc.load_gather(iref, [pos_vec])
        return jnp.where(v >= oh, c64, zero)

      cb_c0 = half(idx_ctx, ctx_rows)
      cb_c1 = half(idx_ctx, ctx_rows + 1)
      cb_c2 = half(idx_ctx, ctx_rows + 2)
      cb_c3 = half(idx_ctx, ctx_rows + 3)
      cb_u = half(idx_cen, bl)
      cb_n0 = half(idx_neg, neg_rows)
      cb_n1 = half(idx_neg, neg_rows + 1)
      cb_n2 = half(idx_neg, neg_rows + 2)
      cb_n3 = half(idx_neg, neg_rows + 3)
      cb_n4 = half(idx_neg, neg_rows + 4)

      def d_body(dq, acc):
        pos_a, n0, n1, n2, n3, n4 = acc
        for q in range(4):
          d = dq * 4 + q
          c0 = plsc.load_gather(rows_ctx, [ctx_rows, cb_c0 + d])
          c1 = plsc.load_gather(rows_ctx, [ctx_rows + 1, cb_c1 + d])
          c2 = plsc.load_gather(rows_ctx, [ctx_rows + 2, cb_c2 + d])
          c3 = plsc.load_gather(rows_ctx, [ctx_rows + 3, cb_c3 + d])
          v = (c0 + c1) + (c2 + c3)
          u = plsc.load_gather(rows_cen, [bl, cb_u + d])
          g0 = plsc.load_gather(rows_neg, [neg_rows, cb_n0 + d])
          g1 = plsc.load_gather(rows_neg, [neg_rows + 1, cb_n1 + d])
          g2 = plsc.load_gather(rows_neg, [neg_rows + 2, cb_n2 + d])
          g3 = plsc.load_gather(rows_neg, [neg_rows + 3, cb_n3 + d])
          g4 = plsc.load_gather(rows_neg, [neg_rows + 4, cb_n4 + d])
          pos_a = pos_a + v * u
          n0 = n0 + v * g0
          n1 = n1 + v * g1
          n2 = n2 + v * g2
          n3 = n3 + v * g3
          n4 = n4 + v * g4
        return pos_a, n0, n1, n2, n3, n4

      z = jnp.zeros((16,), jnp.float32)
      pos_a, n0, n1, n2, n3, n4 = lax.fori_loop(
          0, D // 4, d_body, (z, z, z, z, z, z))

      quarter = jnp.float32(0.25)
      pos_v[pl.ds(g * 16, 16)] = pos_a * quarter
      plsc.store_scatter(neg_v, [neg_rows], n0 * quarter)
      plsc.store_scatter(neg_v, [neg_rows + 1], n1 * quarter)
      plsc.store_scatter(neg_v, [neg_rows + 2], n2 * quarter)
      plsc.store_scatter(neg_v, [neg_rows + 3], n3 * quarter)
      plsc.store_scatter(neg_v, [neg_rows + 4], n4 * quarter)
      return 0

    lax.fori_loop(0, GROUPS, group_body, 0)

    pltpu.sync_copy(pos_v, pos_hbm.at[pl.ds(b0, CHUNK)])
    pltpu.sync_copy(neg_v, neg_hbm.at[pl.ds(b0 * NEG, NNI)])


@jax.jit
def _cbow_sc(ctx_idx, cen_idx, neg_idx, ctx_t, cen_t):
  ctx_emb2 = _pack_table(ctx_t)
  cen_emb2 = _pack_table(cen_t)

  mesh = plsc.VectorSubcoreMesh(core_axis_name="c", subcore_axis_name="s")
  kfn = pl.kernel(
      _body,
      out_type=(
          jax.ShapeDtypeStruct((B,), jnp.float32),
          jax.ShapeDtypeStruct((B * NEG,), jnp.float32),
      ),
      mesh=mesh,
      compiler_params=pltpu.CompilerParams(
          needs_layout_passes=False, use_tc_tiling_on_sc=True),
      scratch_types=[
          pltpu.VMEM((NCI,), jnp.int32),
          pltpu.VMEM((CHUNK,), jnp.int32),
          pltpu.VMEM((NNI,), jnp.int32),
          pltpu.VMEM((NCI,), jnp.int32),
          pltpu.VMEM((CHUNK,), jnp.int32),
          pltpu.VMEM((NNI,), jnp.int32),
          pltpu.VMEM((NCI, 128), jnp.float32),
          pltpu.VMEM((CHUNK, 128), jnp.float32),
          pltpu.VMEM((NNI, 128), jnp.float32),
          pltpu.VMEM((CHUNK,), jnp.float32),
          pltpu.VMEM((NNI,), jnp.float32),
          pltpu.SemaphoreType.DMA,
      ],
  )
  return kfn(ctx_idx, cen_idx, neg_idx, ctx_emb2, cen_emb2)


def kernel(context_words, center_words, negative_samples, context_emb,
           center_emb):
  ctx_idx = context_words.reshape(-1).astype(jnp.int32)
  cen_idx = center_words.astype(jnp.int32)
  neg_idx = negative_samples.reshape(-1).astype(jnp.int32)
  pos, neg = _cbow_sc(ctx_idx, cen_idx, neg_idx, context_emb.T, center_emb.T)
  return pos, neg.reshape(B, NEG)


# speedup vs baseline: 2.6799x; 1.0843x over previous
import jax
import jax.numpy as jnp
from jax import lax
from jax.experimental import pallas as pl
from jax.experimental.pallas import tpu as pltpu
from jax.experimental.pallas import tpu_sc as plsc

VOCAB = 1000000
D = 64
B = 16384
CTX = 4
NEG = 5

NC = 2
NS = 16
NW = NC * NS
B_PER_W = B // NW
CHUNK = 64
NCHUNK = B_PER_W // CHUNK
GROUPS = CHUNK // 16

NCI = CHUNK * CTX
NNI = CHUNK * NEG

VCH = 16384
NBLK = 31
OHALF = NBLK * VCH
LASTB = (VOCAB + VCH - 1) // VCH - 1


def _tc_tr(src_l, src_r, out):
  lt = jnp.swapaxes(src_l[...], 0, 1)
  rt = jnp.swapaxes(src_r[...], 0, 1)
  out[...] = jnp.concatenate([lt, rt], axis=1)


def _pack_table(src_t):
  return pl.pallas_call(
      _tc_tr,
      grid=(NBLK,),
      in_specs=[
          pl.BlockSpec((D, VCH), lambda i: (0, i)),
          pl.BlockSpec((D, VCH), lambda i: (0, jnp.minimum(NBLK + i, LASTB))),
      ],
      out_specs=pl.BlockSpec((VCH, 128), lambda i: (i, 0)),
      out_shape=jax.ShapeDtypeStruct((OHALF, 128), jnp.float32),
  )(src_t, src_t)


def _body_a(ctx_idx_hbm, ctx_emb_hbm, vctx_hbm,
            idx_ctx, pr_ctx, rows_ctx, vctx_v, sem):
  wid = lax.axis_index("s") * NC + lax.axis_index("c")
  base = wid * B_PER_W
  lanes = lax.iota(jnp.int32, 16)
  oh = jnp.int32(OHALF)
  c64 = jnp.int32(64)
  zero = jnp.int32(0)
  quarter = jnp.float32(0.25)

  for c in range(NCHUNK):
    b0 = base + c * CHUNK
    pltpu.sync_copy(ctx_idx_hbm.at[pl.ds(b0 * CTX, NCI)], idx_ctx)

    def sbody(k, _):
      iv = idx_ctx[pl.ds(k * 16, 16)]
      pr_ctx[pl.ds(k * 16, 16)] = jnp.where(iv >= oh, iv - oh, iv)
      return 0
    lax.fori_loop(0, NCI // 16, sbody, 0)

    cps = []
    for k in range(NCI // 128):
      cps.append(pltpu.make_async_copy(
          ctx_emb_hbm.at[pr_ctx.at[pl.ds(k * 128, 128)]],
          rows_ctx.at[pl.ds(k * 128, 128)], sem))
    for cp in cps:
      cp.start()
    for cp in cps:
      cp.wait()

    def group_body(g, _):
      bl = g * 16 + lanes
      ctx_rows = bl * CTX
      vbase = lax.shift_left(bl, 6)

      def half(pos_vec):
        v = plsc.load_gather(idx_ctx, [pos_vec])
        return jnp.where(v >= oh, c64, zero)

      cb_c0 = half(ctx_rows)
      cb_c1 = half(ctx_rows + 1)
      cb_c2 = half(ctx_rows + 2)
      cb_c3 = half(ctx_rows + 3)

      def d_body(dq, _):
        for q in range(4):
          d = dq * 4 + q
          c0 = plsc.load_gather(rows_ctx, [ctx_rows, cb_c0 + d])
          c1 = plsc.load_gather(rows_ctx, [ctx_rows + 1, cb_c1 + d])
          c2 = plsc.load_gather(rows_ctx, [ctx_rows + 2, cb_c2 + d])
          c3 = plsc.load_gather(rows_ctx, [ctx_rows + 3, cb_c3 + d])
          v = ((c0 + c1) + (c2 + c3)) * quarter
          plsc.store_scatter(vctx_v, [vbase + d], v)
        return 0

      lax.fori_loop(0, D // 4, d_body, 0)
      return 0

    lax.fori_loop(0, GROUPS, group_body, 0)
    pltpu.sync_copy(vctx_v, vctx_hbm.at[pl.ds(b0 * D, CHUNK * D)])


def _body_b(cen_idx_hbm, neg_idx_hbm, vctx_hbm, cen_emb_hbm,
            pos_hbm, neg_hbm,
            idx_cen, idx_neg, pr_cen, pr_neg, rows_cen, rows_neg,
            vctx_v, pos_v, neg_v, sem):
  wid = lax.axis_index("s") * NC + lax.axis_index("c")
  base = wid * B_PER_W
  lanes = lax.iota(jnp.int32, 16)
  oh = jnp.int32(OHALF)
  c64 = jnp.int32(64)
  zero = jnp.int32(0)

  for c in range(NCHUNK):
    b0 = base + c * CHUNK
    pltpu.sync_copy(cen_idx_hbm.at[pl.ds(b0, CHUNK)], idx_cen)
    pltpu.sync_copy(neg_idx_hbm.at[pl.ds(b0 * NEG, NNI)], idx_neg)

    def pair_into(dst, srcr, n):
      def sbody(k, _):
        iv = srcr[pl.ds(k * 16, 16)]
        dst[pl.ds(k * 16, 16)] = jnp.where(iv >= oh, iv - oh, iv)
        return 0
      lax.fori_loop(0, n // 16, sbody, 0)
    pair_into(pr_cen, idx_cen, CHUNK)
    pair_into(pr_neg, idx_neg, NNI)

    cps = [pltpu.make_async_copy(
        vctx_hbm.at[pl.ds(b0 * D, CHUNK * D)], vctx_v, sem)]
    cps.append(pltpu.make_async_copy(
        cen_emb_hbm.at[pr_cen], rows_cen, sem))
    for k in range(NNI // 64):
      cps.append(pltpu.make_async_copy(
          cen_emb_hbm.at[pr_neg.at[pl.ds(k * 64, 64)]],
          rows_neg.at[pl.ds(k * 64, 64)], sem))
    for cp in cps:
      cp.start()
    for cp in cps:
      cp.wait()

    def group_body(g, _):
      bl = g * 16 + lanes
      neg_rows = bl * NEG
      vbase = lax.shift_left(bl, 6)

      def half(iref, pos_vec):
        v = plsc.load_gather(iref, [pos_vec])
        return jnp.where(v >= oh, c64, zero)

      cb_u = half(idx_cen, bl)
      cb_n0 = half(idx_neg, neg_rows)
      cb_n1 = half(idx_neg, neg_rows + 1)
      cb_n2 = half(idx_neg, neg_rows + 2)
      cb_n3 = half(idx_neg, neg_rows + 3)
      cb_n4 = half(idx_neg, neg_rows + 4)

      def d_body(dq, acc):
        pos_a, n0, n1, n2, n3, n4 = acc
        for q in range(4):
          d = dq * 4 + q
          v = plsc.load_gather(vctx_v, [vbase + d])
          u = plsc.load_gather(rows_cen, [bl, cb_u + d])
          g0 = plsc.load_gather(rows_neg, [neg_rows, cb_n0 + d])
          g1 = plsc.load_gather(rows_neg, [neg_rows + 1, cb_n1 + d])
          g2 = plsc.load_gather(rows_neg, [neg_rows + 2, cb_n2 + d])
          g3 = plsc.load_gather(rows_neg, [neg_rows + 3, cb_n3 + d])
          g4 = plsc.load_gather(rows_neg, [neg_rows + 4, cb_n4 + d])
          pos_a = pos_a + v * u
          n0 = n0 + v * g0
          n1 = n1 + v * g1
          n2 = n2 + v * g2
          n3 = n3 + v * g3
          n4 = n4 + v * g4
        return pos_a, n0, n1, n2, n3, n4

      z = jnp.zeros((16,), jnp.float32)
      pos_a, n0, n1, n2, n3, n4 = lax.fori_loop(
          0, D // 4, d_body, (z, z, z, z, z, z))

      pos_v[pl.ds(g * 16, 16)] = pos_a
      plsc.store_scatter(neg_v, [neg_rows], n0)
      plsc.store_scatter(neg_v, [neg_rows + 1], n1)
      plsc.store_scatter(neg_v, [neg_rows + 2], n2)
      plsc.store_scatter(neg_v, [neg_rows + 3], n3)
      plsc.store_scatter(neg_v, [neg_rows + 4], n4)
      return 0

    lax.fori_loop(0, GROUPS, group_body, 0)

    pltpu.sync_copy(pos_v, pos_hbm.at[pl.ds(b0, CHUNK)])
    pltpu.sync_copy(neg_v, neg_hbm.at[pl.ds(b0 * NEG, NNI)])


@jax.jit
def _cbow_sc(ctx_idx, cen_idx, neg_idx, ctx_t, cen_t):
  mesh = plsc.VectorSubcoreMesh(core_axis_name="c", subcore_axis_name="s")
  cp = pltpu.CompilerParams(
      needs_layout_passes=False, use_tc_tiling_on_sc=True)

  ctx_emb2 = _pack_table(ctx_t)
  ka = pl.kernel(
      _body_a,
      out_type=jax.ShapeDtypeStruct((B * D,), jnp.float32),
      mesh=mesh,
      compiler_params=cp,
      scratch_types=[
          pltpu.VMEM((NCI,), jnp.int32),
          pltpu.VMEM((NCI,), jnp.int32),
          pltpu.VMEM((NCI, 128), jnp.float32),
          pltpu.VMEM((CHUNK * D,), jnp.float32),
          pltpu.SemaphoreType.DMA,
      ],
  )
  vctx = ka(ctx_idx, ctx_emb2)

  cen_emb2 = _pack_table(cen_t)
  kb = pl.kernel(
      _body_b,
      out_type=(
          jax.ShapeDtypeStruct((B,), jnp.float32),
          jax.ShapeDtypeStruct((B * NEG,), jnp.float32),
      ),
      mesh=mesh,
      compiler_params=cp,
      scratch_types=[
          pltpu.VMEM((CHUNK,), jnp.int32),
          pltpu.VMEM((NNI,), jnp.int32),
          pltpu.VMEM((CHUNK,), jnp.int32),
          pltpu.VMEM((NNI,), jnp.int32),
          pltpu.VMEM((CHUNK, 128), jnp.float32),
          pltpu.VMEM((NNI, 128), jnp.float32),
          pltpu.VMEM((CHUNK * D,), jnp.float32),
          pltpu.VMEM((CHUNK,), jnp.float32),
          pltpu.VMEM((NNI,), jnp.float32),
          pltpu.SemaphoreType.DMA,
      ],
  )
  return kb(cen_idx, neg_idx, vctx, cen_emb2)


def kernel(context_words, center_words, negative_samples, context_emb,
           center_emb):
  ctx_idx = context_words.reshape(-1).astype(jnp.int32)
  cen_idx = center_words.astype(jnp.int32)
  neg_idx = negative_samples.reshape(-1).astype(jnp.int32)
  pos, neg = _cbow_sc(ctx_idx, cen_idx, neg_idx, context_emb.T, center_emb.T)
  return pos, neg.reshape(B, NEG)
